# baseline (speedup 1.0000x reference)
import functools

import jax
import jax.numpy as jnp
import numpy as np
from jax import lax
from jax.experimental import pallas as pl
from jax.experimental.pallas import tpu as pltpu
from jax.experimental.pallas import tpu_sc as plsc

SEQ = 200
DIM = 64
NUM_WORKERS = 32
CHUNK = 128
NBUF = 5
PREFETCH = 2


def _pos_encoding(length, dim):
    pos = np.arange(length)[:, np.newaxis]
    i = np.arange(dim)[np.newaxis, :]
    angle_rates = 1.0 / np.power(10000, 2 * (i // 2) / np.float32(dim))
    angle_rads = pos * angle_rates
    angle_rads[:, 0::2] = np.sin(angle_rads[:, 0::2])
    angle_rads[:, 1::2] = np.cos(angle_rads[:, 1::2])
    return jnp.asarray(angle_rads, dtype=jnp.float32)


def _sc_body(idx_hbm, pos_hbm, table_hbm, out_hbm, idx_v, pos_v, rows_v, t_v,
             *sems):
    gsems, tsems = sems[:NBUF], sems[NBUF:]
    n_chunks = 50
    wid = lax.axis_index("s") * 2 + lax.axis_index("c")
    base = wid * n_chunks

    pltpu.sync_copy(idx_hbm.at[pl.ds(base, n_chunks)], idx_v)
    pltpu.sync_copy(pos_hbm, pos_v)

    lane = lax.iota(jnp.int32, 16)
    row_igs = [lane + 16 * ig for ig in range(CHUNK // 16)]

    def start_gather(j, b):
        pltpu.async_copy(table_hbm.at[idx_v.at[j]], rows_v.at[b], gsems[b])

    for j in range(PREFETCH):
        start_gather(j, j)

    def body(g, carry):
        for b in range(NBUF):
            j = g * NBUF + b
            bn = (b + PREFETCH) % NBUF

            @pl.when(j + PREFETCH < n_chunks)
            def _():
                start_gather(j + PREFETCH, bn)

            pltpu.make_async_copy(
                table_hbm.at[pl.ds(0, CHUNK)], rows_v.at[b], gsems[b]).wait()

            @pl.when(j >= NBUF)
            def _():
                pltpu.make_async_copy(
                    out_hbm.at[0, :, 0], t_v.at[b], tsems[b]).wait()

            c = base + j
            l = 8 * (c // 64) + c % 8
            k = (c // 8) % 8

            rows_b = rows_v.at[b]
            t_b = t_v.at[b]
            l_vec = jnp.broadcast_to(l, (16,))

            def d_body(d, c2, _rows=rows_b, _t=t_b, _lv=l_vec):
                col = jnp.broadcast_to(d, (16,))
                p = plsc.load_gather(pos_v, [_lv, col])
                for ig in range(CHUNK // 16):
                    v = plsc.load_gather(_rows, [row_igs[ig], col])
                    _t[d // 8, d % 8, pl.ds(16 * ig, 16)] = v * 8.0 + p
                return c2

            lax.fori_loop(0, DIM, d_body, 0)

            pltpu.async_copy(t_b, out_hbm.at[l, :, k], tsems[b])
        return carry

    lax.fori_loop(0, n_chunks // NBUF, body, 0)

    for b in range(NBUF):
        pltpu.make_async_copy(
            out_hbm.at[0, :, 0], t_v.at[b], tsems[b]).wait()


def kernel(inputs, table):
    batch, seq = inputs.shape
    vocab, dim = table.shape
    idx = (inputs.T.reshape(seq // 8, 8, batch // CHUNK, CHUNK)
           .transpose(0, 2, 1, 3).reshape(batch * seq // CHUNK, CHUNK))
    pos = _pos_encoding(SEQ, dim)

    mesh = plsc.VectorSubcoreMesh(core_axis_name="c", subcore_axis_name="s")
    f = functools.partial(
        pl.kernel,
        mesh=mesh,
        out_type=jax.ShapeDtypeStruct(
            (seq, dim // 8, batch // CHUNK, 8, CHUNK), jnp.float32),
        compiler_params=pltpu.CompilerParams(
            use_tc_tiling_on_sc=False, needs_layout_passes=False),
        scratch_types=[
            pltpu.VMEM((50, CHUNK), jnp.int32),
            pltpu.VMEM((SEQ, dim), jnp.float32),
            pltpu.VMEM((NBUF, CHUNK, dim), jnp.float32),
            pltpu.VMEM((NBUF, dim // 8, 8, CHUNK), jnp.float32),
        ] + [pltpu.SemaphoreType.DMA] * (2 * NBUF),
    )(_sc_body)
    out = f(idx, pos, table)
    return out.transpose(2, 4, 0, 1, 3).reshape(batch, seq, dim)


# speedup vs baseline: 1.2653x; 1.2653x over previous
import functools

import jax
import jax.numpy as jnp
import numpy as np
from jax import lax
from jax.experimental import pallas as pl
from jax.experimental.pallas import tpu as pltpu
from jax.experimental.pallas import tpu_sc as plsc

SEQ = 200
DIM = 64
NUM_WORKERS = 32
CHUNK = 128
NBUF = 5
PREFETCH = 2
PER_W = SEQ * 8 // NUM_WORKERS


def _pos_encoding(length, dim):
    pos = np.arange(length)[:, np.newaxis]
    i = np.arange(dim)[np.newaxis, :]
    angle_rates = 1.0 / np.power(10000, 2 * (i // 2) / np.float32(dim))
    angle_rads = pos * angle_rates
    angle_rads[:, 0::2] = np.sin(angle_rads[:, 0::2])
    angle_rads[:, 1::2] = np.cos(angle_rads[:, 1::2])
    return jnp.asarray(angle_rads, dtype=jnp.float32)


def _sc_body(tok_hbm, pos_hbm, table_hbm, out_hbm, idx_v, pos_v, rows_v,
             *sems):
    isem = sems[0]
    gsems, wsems = sems[1:1 + NBUF], sems[1 + NBUF:]
    wid = lax.axis_index("s") * 2 + lax.axis_index("c")
    base = wid * PER_W

    for j in range(PER_W):
        c = base + j
        l, k = c // 8, c % 8
        pltpu.async_copy(tok_hbm.at[l, pl.ds(CHUNK * k, CHUNK)],
                         idx_v.at[j], isem)
    pltpu.sync_copy(pos_hbm, pos_v)
    for j in range(PER_W):
        pltpu.make_async_copy(
            tok_hbm.at[0, pl.ds(0, CHUNK)], idx_v.at[j], isem).wait()

    def start_gather(j, b):
        pltpu.async_copy(table_hbm.at[idx_v.at[j]], rows_v.at[b], gsems[b])

    for j in range(PREFETCH):
        start_gather(j, j)

    def body(g, carry):
        for b in range(NBUF):
            j = g * NBUF + b
            bn = (b + PREFETCH) % NBUF

            @pl.when(j + PREFETCH < PER_W)
            def _():
                @pl.when(j + PREFETCH >= NBUF)
                def _():
                    pltpu.make_async_copy(
                        table_hbm.at[pl.ds(0, CHUNK)], rows_v.at[bn],
                        wsems[bn]).wait()
                start_gather(j + PREFETCH, bn)

            pltpu.make_async_copy(
                table_hbm.at[pl.ds(0, CHUNK)], rows_v.at[b], gsems[b]).wait()

            c = base + j
            l, k = c // 8, c % 8
            p0 = pos_v[l, pl.ds(0, 16)]
            p1 = pos_v[l, pl.ds(16, 16)]
            p2 = pos_v[l, pl.ds(32, 16)]
            p3 = pos_v[l, pl.ds(48, 16)]

            def row_body(r, c2, _b=b, _p=(p0, p1, p2, p3)):
                for q in range(DIM // 16):
                    sl = pl.ds(16 * q, 16)
                    rows_v[_b, r, sl] = rows_v[_b, r, sl] * 8.0 + _p[q]
                return c2

            lax.fori_loop(0, CHUNK, row_body, 0, unroll=4)

            pltpu.async_copy(
                rows_v.at[b], out_hbm.at[pl.ds(CHUNK * k, CHUNK), l],
                wsems[b])
        return carry

    lax.fori_loop(0, PER_W // NBUF, body, 0)

    for b in range(NBUF):
        pltpu.make_async_copy(
            table_hbm.at[pl.ds(0, CHUNK)], rows_v.at[b], wsems[b]).wait()


def kernel(inputs, table):
    batch, seq = inputs.shape
    vocab, dim = table.shape
    pos = _pos_encoding(SEQ, dim)

    mesh = plsc.VectorSubcoreMesh(core_axis_name="c", subcore_axis_name="s")
    f = functools.partial(
        pl.kernel,
        mesh=mesh,
        out_type=jax.ShapeDtypeStruct((batch, seq, dim), jnp.float32),
        compiler_params=pltpu.CompilerParams(
            use_tc_tiling_on_sc=False, needs_layout_passes=False),
        scratch_types=[
            pltpu.VMEM((PER_W, CHUNK), jnp.int32),
            pltpu.VMEM((SEQ, dim), jnp.float32),
            pltpu.VMEM((NBUF, CHUNK, dim), jnp.float32),
        ] + [pltpu.SemaphoreType.DMA] * (1 + 2 * NBUF),
    )(_sc_body)
    return f(inputs.T, pos, table)


# speedup vs baseline: 1.2769x; 1.0092x over previous
import functools

import jax
import jax.numpy as jnp
import numpy as np
from jax import lax
from jax.experimental import pallas as pl
from jax.experimental.pallas import tpu as pltpu
from jax.experimental.pallas import tpu_sc as plsc

SEQ = 200
DIM = 64
NUM_WORKERS = 32
CHUNK = 128
PITCH = 129
NBUF = 5
PREFETCH = 2
PER_W = SEQ * 8 // NUM_WORKERS
N_TILES = SEQ * 8 // 8


def _pos_encoding(length, dim):
    pos = np.arange(length)[:, np.newaxis]
    i = np.arange(dim)[np.newaxis, :]
    angle_rates = 1.0 / np.power(10000, 2 * (i // 2) / np.float32(dim))
    angle_rads = pos * angle_rates
    angle_rads[:, 0::2] = np.sin(angle_rads[:, 0::2])
    angle_rads[:, 1::2] = np.cos(angle_rads[:, 1::2])
    return jnp.asarray(angle_rads, dtype=jnp.float32)


def _reblock_body(tok_hbm, idx_hbm):
    wid = lax.axis_index("s") * 2 + lax.axis_index("c")
    for i in range((N_TILES + NUM_WORKERS - 1) // NUM_WORKERS):
        t = wid + NUM_WORKERS * i

        @pl.when(t < N_TILES)
        def _():
            tl, k = t // 8, t % 8
            pltpu.sync_copy(
                tok_hbm.at[pl.ds(8 * tl, 8), pl.ds(CHUNK * k, CHUNK)],
                idx_hbm.at[pl.ds(8 * t, 8), :])


def _sc_body(idx_hbm, pos_hbm, table_hbm, out_hbm, idx_v, pos_v, rows_v, t_v,
             *sems):
    gsems, wsems = sems[:NBUF], sems[NBUF:]
    wid = lax.axis_index("s") * 2 + lax.axis_index("c")
    base = wid * PER_W

    pltpu.sync_copy(idx_hbm.at[pl.ds(base, PER_W)], idx_v)
    pltpu.sync_copy(pos_hbm, pos_v)

    lane = lax.iota(jnp.int32, 16)
    row_igs = [lane + 16 * ig for ig in range(CHUNK // 16)]

    a_qs = [(lane + 16 * q) // 8 for q in range(DIM // 16)]
    b_qs = [(lane + 16 * q) % 8 for q in range(DIM // 16)]

    def start_gather(j, b):
        pltpu.async_copy(table_hbm.at[idx_v.at[j]], rows_v.at[b], gsems[b])

    for j in range(PREFETCH):
        start_gather(j, j)

    def body(g, carry):
        for b in range(NBUF):
            j = g * NBUF + b
            bn = (b + PREFETCH) % NBUF

            @pl.when(j + PREFETCH < PER_W)
            def _():
                @pl.when(j + PREFETCH >= NBUF)
                def _():
                    pltpu.make_async_copy(
                        table_hbm.at[pl.ds(0, CHUNK)], rows_v.at[bn],
                        wsems[bn]).wait()
                start_gather(j + PREFETCH, bn)

            pltpu.make_async_copy(
                table_hbm.at[pl.ds(0, CHUNK)], rows_v.at[b], gsems[b]).wait()

            c = base + j
            l = 8 * (c // 64) + c % 8
            k = (c // 8) % 8
            p0 = pos_v[l, pl.ds(0, 16)]
            p1 = pos_v[l, pl.ds(16, 16)]
            p2 = pos_v[l, pl.ds(32, 16)]
            p3 = pos_v[l, pl.ds(48, 16)]

            t_b = t_v.at[b]

            def row_body(i, c2, _b=b, _t=t_b, _p=(p0, p1, p2, p3)):
                col_i = jnp.broadcast_to(i, (16,))
                for q in range(DIM // 16):
                    v = rows_v[_b, i, pl.ds(16 * q, 16)]
                    plsc.store_scatter(_t, [a_qs[q], b_qs[q], col_i],
                                       v * 8.0 + _p[q])
                return c2

            lax.fori_loop(0, CHUNK, row_body, 0, unroll=2)

            pltpu.async_copy(t_b.at[:, :, pl.ds(0, CHUNK)],
                             out_hbm.at[l, :, k], wsems[b])
        return carry

    lax.fori_loop(0, PER_W // NBUF, body, 0)

    for b in range(NBUF):
        pltpu.make_async_copy(
            table_hbm.at[pl.ds(0, CHUNK)], rows_v.at[b], wsems[b]).wait()


def kernel(inputs, table):
    batch, seq = inputs.shape
    vocab, dim = table.shape
    pos = _pos_encoding(SEQ, dim)

    mesh = plsc.VectorSubcoreMesh(core_axis_name="c", subcore_axis_name="s")

    reblock = functools.partial(
        pl.kernel,
        mesh=mesh,
        out_type=jax.ShapeDtypeStruct((batch * seq // CHUNK, CHUNK),
                                      jnp.int32),
        compiler_params=pltpu.CompilerParams(
            use_tc_tiling_on_sc=True, needs_layout_passes=False),
    )(_reblock_body)
    idx = reblock(inputs.T)

    f = functools.partial(
        pl.kernel,
        mesh=mesh,
        out_type=jax.ShapeDtypeStruct(
            (seq, dim // 8, batch // CHUNK, 8, CHUNK), jnp.float32),
        compiler_params=pltpu.CompilerParams(
            use_tc_tiling_on_sc=False, needs_layout_passes=False),
        scratch_types=[
            pltpu.VMEM((PER_W, CHUNK), jnp.int32),
            pltpu.VMEM((SEQ, dim), jnp.float32),
            pltpu.VMEM((NBUF, CHUNK, dim), jnp.float32),
            pltpu.VMEM((NBUF, dim // 8, 8, PITCH), jnp.float32),
        ] + [pltpu.SemaphoreType.DMA] * (2 * NBUF),
    )(_sc_body)
    out = f(idx, pos, table)
    return out.transpose(2, 4, 0, 1, 3).reshape(batch, seq, dim)
